# baseline (speedup 1.0000x reference)
import jax
import jax.numpy as jnp
from jax.experimental import pallas as pl
from jax.experimental.pallas import tpu as pltpu


def _fwd_kernel(ids_ref, emb_ref, w1_ref, b1bc_ref, w2a_ref, w3a_ref,
                out_ref, t_ref):
    f32 = jnp.float32
    bf16 = jnp.bfloat16
    nf, tb = ids_ref.shape
    v, d = emb_ref.shape
    h = w1_ref.shape[0]

    @pl.when(pl.program_id(0) == 0)
    def _precompute_tables():
        emb = emb_ref[...]
        for i in range(nf):
            w1_i = w1_ref[:, i * d:(i + 1) * d]
            tt = jax.lax.dot_general(w1_i, emb, (((1,), (1,)), ((), ())),
                                     preferred_element_type=f32)
            if i == 0:
                tt = tt + b1bc_ref[...]
            t_ref[:, i * v:(i + 1) * v] = tt.astype(bf16)

    ids = ids_ref[...]
    sub_iota = jax.lax.broadcasted_iota(jnp.int32, (v, tb), 0)
    oh = jnp.concatenate(
        [(ids[i:i + 1, :] == sub_iota).astype(bf16)
         for i in range(nf)], axis=0)
    x = jax.lax.dot_general(t_ref[...], oh, (((1,), (0,)), ((), ())),
                            preferred_element_type=f32)

    alpha = 1.6732632423543772
    scale = 1.0507009873554805
    h1 = scale * jnp.where(x > 0, x, alpha * (jnp.exp(x) - 1.0))
    ones_row = jnp.ones((1, tb), f32)
    h1a = jnp.concatenate([h1, ones_row], axis=0)
    h2 = jnp.tanh(
        jax.lax.dot_general(w2a_ref[...], h1a, (((1,), (0,)), ((), ())),
                            preferred_element_type=f32))
    h2a = jnp.concatenate([h2, ones_row], axis=0)
    o8 = jax.lax.dot_general(w3a_ref[...], h2a, (((1,), (0,)), ((), ())),
                             preferred_element_type=f32)
    out_ref[...] = o8[0:1, :]


def kernel(vocab_ids, pos_emb, W1, b1, W2, b2, W3, b3):
    nf, b = vocab_ids.shape
    v, d = pos_emb.shape
    h = W1.shape[0]
    ids = vocab_ids.astype(jnp.int32)
    b1bc = jnp.broadcast_to(b1[:, None], (h, v))
    w2a = jnp.concatenate([W2, b2[:, None]], axis=1)
    w3a = jnp.pad(jnp.concatenate([W3, b3[:, None]], axis=1),
                  ((0, 7), (0, 0)))
    tb = 2048 if b % 2048 == 0 else b
    nb = b // tb
    out_row = pl.pallas_call(
        _fwd_kernel,
        grid=(nb,),
        in_specs=[
            pl.BlockSpec((nf, tb), lambda i: (0, i)),
            pl.BlockSpec(pos_emb.shape, lambda i: (0, 0)),
            pl.BlockSpec(W1.shape, lambda i: (0, 0)),
            pl.BlockSpec((h, v), lambda i: (0, 0)),
            pl.BlockSpec(w2a.shape, lambda i: (0, 0)),
            pl.BlockSpec(w3a.shape, lambda i: (0, 0)),
        ],
        out_specs=pl.BlockSpec((1, tb), lambda i: (0, i)),
        out_shape=jax.ShapeDtypeStruct((1, b), jnp.float32),
        scratch_shapes=[pltpu.VMEM((h, nf * v), jnp.bfloat16)],
    )(ids, pos_emb, W1, b1bc, w2a, w3a)
    return out_row.reshape(b, 1)


# speedup vs baseline: 1.0831x; 1.0831x over previous
import jax
import jax.numpy as jnp
from jax.experimental import pallas as pl
from jax.experimental.pallas import tpu as pltpu


def _fwd_kernel(ids_ref, emb_ref, w1_ref, b1bc_ref, w2a_ref, w3a_ref,
                out_ref, t_ref):
    f32 = jnp.float32
    bf16 = jnp.bfloat16
    nf, tb = ids_ref.shape
    v, d = emb_ref.shape
    h = w1_ref.shape[0]

    @pl.when(pl.program_id(0) == 0)
    def _precompute_tables():
        emb = emb_ref[...]
        for i in range(nf):
            w1_i = w1_ref[:, i * d:(i + 1) * d]
            tt = jax.lax.dot_general(w1_i, emb, (((1,), (1,)), ((), ())),
                                     preferred_element_type=f32)
            if i == 0:
                tt = tt + b1bc_ref[...]
            t_ref[:, i * v:(i + 1) * v] = tt.astype(bf16)

    ids = ids_ref[...]
    sub_iota = jax.lax.broadcasted_iota(jnp.int32, (v, tb), 0)
    oh = jnp.concatenate(
        [(ids[i:i + 1, :] == sub_iota).astype(bf16)
         for i in range(nf)], axis=0)
    x = jax.lax.dot_general(t_ref[...], oh, (((1,), (0,)), ((), ())),
                            preferred_element_type=f32)

    alpha = 1.6732632423543772
    scale = 1.0507009873554805
    h1 = scale * jnp.where(x > 0, x, alpha * (jnp.exp(x) - 1.0))
    ones_row = jnp.ones((1, tb), f32)
    h1a = jnp.concatenate([h1, ones_row], axis=0)
    h2 = jnp.tanh(
        jax.lax.dot_general(w2a_ref[...], h1a, (((1,), (0,)), ((), ())),
                            preferred_element_type=f32))
    h2a = jnp.concatenate([h2, ones_row], axis=0)
    o8 = jax.lax.dot_general(w3a_ref[...], h2a, (((1,), (0,)), ((), ())),
                             preferred_element_type=f32)
    out_ref[...] = o8[0:1, :]


def kernel(vocab_ids, pos_emb, W1, b1, W2, b2, W3, b3):
    nf, b = vocab_ids.shape
    v, d = pos_emb.shape
    h = W1.shape[0]
    ids = vocab_ids.astype(jnp.int32)
    b1bc = jnp.broadcast_to(b1[:, None], (h, v))
    w2a = jnp.concatenate([W2, b2[:, None]], axis=1)
    w3a = jnp.pad(jnp.concatenate([W3, b3[:, None]], axis=1),
                  ((0, 7), (0, 0)))
    tb = 8192 if b % 8192 == 0 else b
    nb = b // tb
    out_row = pl.pallas_call(
        _fwd_kernel,
        grid=(nb,),
        in_specs=[
            pl.BlockSpec((nf, tb), lambda i: (0, i)),
            pl.BlockSpec(pos_emb.shape, lambda i: (0, 0)),
            pl.BlockSpec(W1.shape, lambda i: (0, 0)),
            pl.BlockSpec((h, v), lambda i: (0, 0)),
            pl.BlockSpec(w2a.shape, lambda i: (0, 0)),
            pl.BlockSpec(w3a.shape, lambda i: (0, 0)),
        ],
        out_specs=pl.BlockSpec((1, tb), lambda i: (0, i)),
        out_shape=jax.ShapeDtypeStruct((1, b), jnp.float32),
        scratch_shapes=[pltpu.VMEM((h, nf * v), jnp.bfloat16)],
    )(ids, pos_emb, W1, b1bc, w2a, w3a)
    return out_row.reshape(b, 1)


# speedup vs baseline: 1.0859x; 1.0026x over previous
import jax
import jax.numpy as jnp
from jax.experimental import pallas as pl
from jax.experimental.pallas import tpu as pltpu


def _fwd_kernel(ids_ref, emb_ref, w1_ref, b1bc_ref, w2a_ref, w3a_ref,
                out_ref, t_ref):
    f32 = jnp.float32
    bf16 = jnp.bfloat16
    nf, tb = ids_ref.shape
    v, d = emb_ref.shape
    h = w1_ref.shape[0]

    @pl.when(pl.program_id(0) == 0)
    def _precompute_tables():
        emb = emb_ref[...]
        for i in range(nf):
            w1_i = w1_ref[:, i * d:(i + 1) * d]
            tt = jax.lax.dot_general(w1_i, emb, (((1,), (1,)), ((), ())),
                                     preferred_element_type=f32)
            if i == 0:
                tt = tt + b1bc_ref[...]
            t_ref[:, i * v:(i + 1) * v] = tt.astype(bf16)

    ids = ids_ref[...]
    sub_iota = jax.lax.broadcasted_iota(jnp.int32, (v, tb), 0)
    oh = jnp.concatenate(
        [(ids[i:i + 1, :] == sub_iota).astype(bf16)
         for i in range(nf)], axis=0)
    x = jax.lax.dot_general(t_ref[...], oh, (((1,), (0,)), ((), ())),
                            preferred_element_type=f32)

    alpha = 1.6732632423543772
    scale = 1.0507009873554805
    h1 = scale * jnp.where(x > 0, x, alpha * (jnp.exp(x) - 1.0))
    ones_row = jnp.ones((1, tb), f32)
    h1a = jnp.concatenate([h1, ones_row], axis=0)
    h2 = jnp.tanh(
        jax.lax.dot_general(w2a_ref[...], h1a, (((1,), (0,)), ((), ())),
                            preferred_element_type=f32))
    h2a = jnp.concatenate([h2, ones_row], axis=0)
    o8 = jax.lax.dot_general(w3a_ref[...], h2a, (((1,), (0,)), ((), ())),
                             preferred_element_type=f32)
    out_ref[...] = o8[0:1, :]


def kernel(vocab_ids, pos_emb, W1, b1, W2, b2, W3, b3):
    nf, b = vocab_ids.shape
    v, d = pos_emb.shape
    h = W1.shape[0]
    ids = vocab_ids.astype(jnp.int32)
    b1bc = jnp.broadcast_to(b1[:, None], (h, v))
    w2a = jnp.concatenate([W2, b2[:, None]], axis=1)
    w3a = jnp.pad(jnp.concatenate([W3, b3[:, None]], axis=1),
                  ((0, 7), (0, 0)))
    tb = b
    nb = b // tb
    out_row = pl.pallas_call(
        _fwd_kernel,
        grid=(nb,),
        in_specs=[
            pl.BlockSpec((nf, tb), lambda i: (0, i)),
            pl.BlockSpec(pos_emb.shape, lambda i: (0, 0)),
            pl.BlockSpec(W1.shape, lambda i: (0, 0)),
            pl.BlockSpec((h, v), lambda i: (0, 0)),
            pl.BlockSpec(w2a.shape, lambda i: (0, 0)),
            pl.BlockSpec(w3a.shape, lambda i: (0, 0)),
        ],
        out_specs=pl.BlockSpec((1, tb), lambda i: (0, i)),
        out_shape=jax.ShapeDtypeStruct((1, b), jnp.float32),
        scratch_shapes=[pltpu.VMEM((h, nf * v), jnp.bfloat16)],
    )(ids, pos_emb, W1, b1bc, w2a, w3a)
    return out_row.reshape(b, 1)
